# baseline (speedup 1.0000x reference)
import functools

import jax
import jax.numpy as jnp
from jax import lax
from jax.experimental import pallas as pl
from jax.experimental.pallas import tpu as pltpu
from jax.experimental.pallas import tpu_sc as plsc

VOCAB = 100000
DIM = 300
BATCH = 16384

_NC = 2
_NS = 16
_NW = _NC * _NS
_NT_FULL = VOCAB // 128
_REM = VOCAB - _NT_FULL * 128
_NT_LO = _NT_FULL // _NW
_NT_HI = _NT_LO + 1
_N_HI = _NT_FULL - _NT_LO * _NW
_SEG = 1024
_SH = BATCH + 128
_SW = 384
_FR = 32
_NJ = 19

_mesh = plsc.VectorSubcoreMesh(core_axis_name="c", subcore_axis_name="s")


@functools.partial(
    pl.kernel,
    mesh=_mesh,
    out_type=jax.ShapeDtypeStruct((_SH, _SW), jnp.float32),
    compiler_params=pltpu.CompilerParams(needs_layout_passes=False),
    scratch_types=[
        pltpu.VMEM((2048,), jnp.int32),
        pltpu.VMEM((BATCH + 64,), jnp.int32),
        pltpu.VMEM((DIM, 128), jnp.float32),
        pltpu.VMEM((DIM, _REM), jnp.float32),
        pltpu.VMEM((_SEG + 16,), jnp.int32),
        pltpu.VMEM((_FR, _SW), jnp.float32),
        pltpu.VMEM((_FR,), jnp.int32),
        pltpu.SemaphoreType.DMA,
    ],
)
def _sc_gather(ids, tabT, out_hbm,
               ids_v, plist, slab_v, slab32, pseg, outbuf, bbuf, sem_f):
    wid = lax.axis_index("s") * _NC + lax.axis_index("c")
    iota = lax.iota(jnp.int32, 16)
    dumpvec = BATCH + (wid % 8) * 16 + iota
    allm = jnp.ones((16,), jnp.bool_)

    for q in range(_FR // 16):
        bbuf[pl.ds(q * 16, 16)] = dumpvec

    def scan_ids(w0, w1):
        cnt = 0
        for c in range(BATCH // 2048):
            pltpu.sync_copy(ids.at[pl.ds(c * 2048, 2048)], ids_v)

            def scan_body(i, cnt, _c=c):
                vv = ids_v[pl.ds(i * 16, 16)]
                m = (vv >= w0) & (vv < w1)
                bb = iota + (_c * 2048 + i * 16)
                pk = lax.shift_left(vv, 15) | bb
                pos = cnt + plsc.cumsum(m.astype(jnp.int32)) - 1
                plsc.store_scatter(plist, [pos], pk, mask=m)
                return cnt + plsc.all_reduce_population_count(m)[0]

            cnt = lax.fori_loop(0, 128, scan_body, cnt)
        plsc.store_scatter(plist, [cnt + iota],
                           lax.shift_left(jnp.full((16,), w0, jnp.int32), 15)
                           | dumpvec, mask=allm)
        return cnt

    def do_flush():
        pltpu.async_copy(outbuf, out_hbm.at[bbuf], sem_f).wait()

    def make_select(sl_ref):
        def select_group(g2, fill):
            pp = pseg[pl.ds(g2 * 16, 16)]
            vv = lax.shift_right_logical(pp, 15)
            bb = pp & 32767
            fm = fill % _FR
            bbuf[pl.ds(fm, 16)] = bb
            for k in range(16):
                v = vv[k]
                orow = fm + k
                cols = jnp.full((16,), v, jnp.int32)
                for j in range(_NJ):
                    lo = j * 16 if j < _NJ - 1 else DIM - 16
                    rows = iota + lo
                    g16 = plsc.load_gather(sl_ref, [rows, cols])
                    outbuf[orow, pl.ds(lo, 16)] = g16
            fill = fill + 16

            @pl.when(fill % _FR == 0)
            def _():
                do_flush()

            return fill
        return select_group

    def process_window(sl_ref, o, width, cnt, fill):
        sel = make_select(sl_ref)
        o15 = lax.shift_left(o, 15)

        def seg_loop(carry):
            p, fill = carry
            pe = jnp.minimum(p + _SEG, cnt)

            def seg_scan(g, scnt):
                pp = plist[pl.ds(p + g * 16, 16)]
                vv = lax.shift_right_logical(pp, 15)
                m = (vv >= o) & (vv < o + width)
                pk = pp - o15
                pos = scnt + plsc.cumsum(m.astype(jnp.int32)) - 1
                plsc.store_scatter(pseg, [pos], pk, mask=m)
                return scnt + plsc.all_reduce_population_count(m)[0]

            ng = (pe - p + 15) // 16
            scnt = lax.fori_loop(0, ng, seg_scan, 0)
            plsc.store_scatter(pseg, [scnt + iota], dumpvec, mask=allm)
            fill = lax.fori_loop(0, (scnt + 15) // 16, sel, fill)
            return (pe, fill)

        _, fill = lax.while_loop(lambda c: c[0] < cnt, seg_loop, (0, fill))
        return fill

    def pad_and_flush(fill):
        f3 = lax.while_loop(
            lambda f: f % _FR != 0,
            lambda f: (bbuf.__setitem__(pl.ds(f % _FR, 16), dumpvec),
                       f + 16)[1],
            fill)
        del f3
        do_flush()

    ntiles = jnp.where(wid < _N_HI, _NT_HI, _NT_LO)
    tile0 = wid * _NT_LO + jnp.minimum(wid, _N_HI)
    w0 = tile0 * 128
    w1 = w0 + ntiles * 128
    cnt = scan_ids(w0, w1)

    def slab_body(s, fill):
        o = pl.multiple_of((tile0 + s) * 128, 128)
        pltpu.sync_copy(tabT.at[:, pl.ds(o, 128)], slab_v)
        return process_window(slab_v, o, 128, cnt, fill)

    fill = lax.fori_loop(0, ntiles, slab_body, 0)

    @pl.when(wid == _NW - 1)
    def _():
        cnt2 = scan_ids(_NT_FULL * 128, VOCAB)
        pltpu.sync_copy(tabT.at[:, pl.ds(_NT_FULL * 128, _REM)], slab32)
        fill2 = process_window(slab32, _NT_FULL * 128, _REM, cnt2, fill)
        pad_and_flush(fill2)

    @pl.when(wid != _NW - 1)
    def _():
        pad_and_flush(fill)


_BLK = 2048
_NTOP = BATCH // _BLK
_NBLK = 2 * _NTOP


def _tc_body(xs_ref, xt_ref, we_ref, b_ref, wd_ref, o_ref, wc_ref, bc_ref):
    i = pl.program_id(0)

    @pl.when(i == 0)
    def _():
        wc_ref[...] = lax.dot_general(
            we_ref[...], wd_ref[...], (((0,), (0,)), ((), ())),
            preferred_element_type=jnp.float32)
        bc_ref[...] = lax.dot_general(
            wd_ref[...], b_ref[...], (((0,), (1,)), ((), ())),
            preferred_element_type=jnp.float32)

    @pl.when(i < _NTOP)
    def _():
        o_ref[...] = lax.dot_general(
            wc_ref[...], xs_ref[:, :DIM], (((0,), (1,)), ((), ())),
            preferred_element_type=jnp.float32) + bc_ref[...]

    @pl.when(i >= _NTOP)
    def _():
        o_ref[...] = xt_ref[:, :DIM].T


def _tc_transform(stag_src, stag_tgt, W_enc, b2, W_dec):
    return pl.pallas_call(
        _tc_body,
        grid=(_NBLK,),
        in_specs=[
            pl.BlockSpec((_BLK, _SW), lambda i: (jnp.minimum(i, _NTOP - 1), 0)),
            pl.BlockSpec((_BLK, _SW),
                         lambda i: (jnp.maximum(i - _NTOP, 0), 0)),
            pl.BlockSpec((DIM, DIM), lambda i: (0, 0)),
            pl.BlockSpec((1, DIM), lambda i: (0, 0)),
            pl.BlockSpec((DIM, DIM), lambda i: (0, 0)),
        ],
        out_specs=pl.BlockSpec((DIM, _BLK), lambda i: (0, i)),
        out_shape=jax.ShapeDtypeStruct((DIM, 2 * BATCH), jnp.float32),
        scratch_shapes=[
            pltpu.VMEM((DIM, DIM), jnp.float32),
            pltpu.VMEM((DIM, 1), jnp.float32),
        ],
    )(stag_src, stag_tgt, W_enc, b2, W_dec)


def kernel(src_id, tgt_id, emb_src, emb_tgt, W_enc, b_enc, W_dec):
    src_id = src_id.astype(jnp.int32)
    tgt_id = tgt_id.astype(jnp.int32)
    stag_src = _sc_gather(src_id, emb_src.T)
    stag_tgt = _sc_gather(tgt_id, emb_tgt.T)
    b2 = b_enc.reshape(1, DIM)
    out_t = _tc_transform(stag_src, stag_tgt, W_enc, b2, W_dec)
    return out_t.T


# speedup vs baseline: 1.6680x; 1.6680x over previous
import functools

import jax
import jax.numpy as jnp
from jax import lax
from jax.experimental import pallas as pl
from jax.experimental.pallas import tpu as pltpu
from jax.experimental.pallas import tpu_sc as plsc

VOCAB = 100000
DIM = 300
BATCH = 16384

_NC = 2
_NS = 16
_NW = _NC * _NS
_BPW = BATCH // _NW
_CH = 128
_NCH = _BPW // _CH
_KG = 16
_NG = _CH // _KG
_MAIN = 256
_TAIL = DIM - _MAIN
_SW = 384

_mesh = plsc.VectorSubcoreMesh(core_axis_name="c", subcore_axis_name="s")


@functools.partial(
    pl.kernel,
    mesh=_mesh,
    out_type=jax.ShapeDtypeStruct((BATCH, _SW), jnp.float32),
    scratch_types=[
        pltpu.VMEM((_CH,), jnp.int32),
        pltpu.VMEM((_CH, _MAIN), jnp.float32),
        pltpu.VMEM((_KG, 8, _TAIL), jnp.float32),
        pltpu.VMEM((_CH, 128), jnp.float32),
        pltpu.SemaphoreType.DMA,
        pltpu.SemaphoreType.DMA,
    ],
)
def _sc_gather(ids, tab, out_hbm, idx_v, main_v, blk_v, tail_v, sem_m, sem_t):
    wid = lax.axis_index("s") * _NC + lax.axis_index("c")
    base = wid * _BPW
    for j in range(_NCH):
        pltpu.sync_copy(ids.at[pl.ds(base + j * _CH, _CH)], idx_v)
        main_cp = pltpu.async_copy(
            tab.at[idx_v, pl.ds(0, _MAIN)], main_v, sem_m)

        def tail_group(g, _):
            gidx = idx_v[pl.ds(g * _KG, _KG)]
            for k in range(_KG):
                idx = gidx[k]
                i8 = pl.multiple_of((idx // 8) * 8, 8)
                pltpu.async_copy(
                    tab.at[pl.ds(i8, 8), pl.ds(_MAIN, _TAIL)],
                    blk_v.at[k], sem_t)
            for k in range(_KG):
                pltpu.make_async_copy(
                    tab.at[pl.ds(0, 8), pl.ds(_MAIN, _TAIL)],
                    blk_v.at[k], sem_t).wait()
            for k in range(_KG):
                idx = gidx[k]
                r = lax.rem(idx, 8)
                row = g * _KG + k
                tail_v[row, pl.ds(0, 16)] = blk_v[k, r, pl.ds(0, 16)]
                tail_v[row, pl.ds(16, 16)] = blk_v[k, r, pl.ds(16, 16)]
                tail_v[row, pl.ds(_TAIL - 16, 16)] = (
                    blk_v[k, r, pl.ds(_TAIL - 16, 16)])
            return 0

        lax.fori_loop(0, _NG, tail_group, 0)
        main_cp.wait()
        ob = base + j * _CH
        pltpu.sync_copy(main_v, out_hbm.at[pl.ds(ob, _CH), pl.ds(0, _MAIN)])
        pltpu.sync_copy(tail_v,
                        out_hbm.at[pl.ds(ob, _CH), pl.ds(_MAIN, 128)])


_BLK = 2048
_NTOP = BATCH // _BLK
_NBLK = 2 * _NTOP


def _tc_src_body(xs_ref, we_ref, b_ref, wd_ref, o_ref, wc_ref, bc_ref):
    i = pl.program_id(0)

    @pl.when(i == 0)
    def _():
        wc_ref[...] = lax.dot_general(
            we_ref[...], wd_ref[...], (((0,), (0,)), ((), ())),
            preferred_element_type=jnp.float32)
        bc_ref[...] = lax.dot_general(
            wd_ref[...], b_ref[...], (((0,), (1,)), ((), ())),
            preferred_element_type=jnp.float32)

    o_ref[...] = lax.dot_general(
        wc_ref[...], xs_ref[:, :DIM], (((0,), (1,)), ((), ())),
        preferred_element_type=jnp.float32) + bc_ref[...]


def _tc_src(stag_src, W_enc, b2, W_dec):
    return pl.pallas_call(
        _tc_src_body,
        grid=(_NTOP,),
        in_specs=[
            pl.BlockSpec((_BLK, _SW), lambda i: (i, 0)),
            pl.BlockSpec((DIM, DIM), lambda i: (0, 0)),
            pl.BlockSpec((1, DIM), lambda i: (0, 0)),
            pl.BlockSpec((DIM, DIM), lambda i: (0, 0)),
        ],
        out_specs=pl.BlockSpec((DIM, _BLK), lambda i: (0, i)),
        out_shape=jax.ShapeDtypeStruct((DIM, 2 * BATCH), jnp.float32),
        scratch_shapes=[
            pltpu.VMEM((DIM, DIM), jnp.float32),
            pltpu.VMEM((DIM, 1), jnp.float32),
        ],
    )(stag_src, W_enc, b2, W_dec)


def _tc_tgt_body(o_in_ref, xt_ref, o_ref):
    del o_in_ref
    o_ref[...] = xt_ref[:, :DIM].T


def _tc_tgt(out_t, stag_tgt):
    return pl.pallas_call(
        _tc_tgt_body,
        grid=(_NTOP,),
        in_specs=[
            pl.BlockSpec((8, 128), lambda i: (0, 0)),
            pl.BlockSpec((_BLK, _SW), lambda i: (i, 0)),
        ],
        out_specs=pl.BlockSpec((DIM, _BLK), lambda i: (0, i + _NTOP)),
        out_shape=jax.ShapeDtypeStruct((DIM, 2 * BATCH), jnp.float32),
        input_output_aliases={0: 0},
    )(out_t, stag_tgt)


def kernel(src_id, tgt_id, emb_src, emb_tgt, W_enc, b_enc, W_dec):
    src_id = src_id.astype(jnp.int32)
    tgt_id = tgt_id.astype(jnp.int32)
    stag_src = _sc_gather(src_id, emb_src)
    stag_tgt = _sc_gather(tgt_id, emb_tgt)
    b2 = b_enc.reshape(1, DIM)
    out_t = _tc_src(stag_src, W_enc, b2, W_dec)
    out_t = _tc_tgt(out_t, stag_tgt)
    return out_t.T


# speedup vs baseline: 1.6887x; 1.0124x over previous
import functools

import jax
import jax.numpy as jnp
from jax import lax
from jax.experimental import pallas as pl
from jax.experimental.pallas import tpu as pltpu
from jax.experimental.pallas import tpu_sc as plsc

VOCAB = 100000
DIM = 300
BATCH = 16384

_NC = 2
_NS = 16
_NW = _NC * _NS
_BPW = BATCH // _NW
_CH = 128
_NCH = _BPW // _CH
_KG = 16
_NG = _CH // _KG
_MAIN = 256
_TAIL = DIM - _MAIN
_SW = 384

_mesh = plsc.VectorSubcoreMesh(core_axis_name="c", subcore_axis_name="s")


@functools.partial(
    pl.kernel,
    mesh=_mesh,
    out_type=jax.ShapeDtypeStruct((BATCH, _SW), jnp.float32),
    scratch_types=[
        pltpu.VMEM((_CH,), jnp.int32),
        pltpu.VMEM((_CH, _SW), jnp.float32),
        pltpu.VMEM((2, _KG, 8, _TAIL), jnp.float32),
        pltpu.SemaphoreType.DMA,
        pltpu.SemaphoreType.DMA,
        pltpu.SemaphoreType.DMA,
        pltpu.SemaphoreType.DMA,
    ],
)
def _sc_gather(ids, tab, out_hbm, idx_v, row_v, blk_v,
               sem_m, sem_w, sem_t0, sem_t1):
    wid = lax.axis_index("s") * _NC + lax.axis_index("c")
    base = wid * _BPW

    def fire_tail(g, h):
        gidx = idx_v[pl.ds(g * _KG, _KG)]
        for k in range(_KG):
            idx = gidx[k]
            i8 = pl.multiple_of((idx // 8) * 8, 8)
            pltpu.async_copy(
                tab.at[pl.ds(i8, 8), pl.ds(_MAIN, _TAIL)],
                blk_v.at[h, k], sem_t0 if h == 0 else sem_t1)

    def drain_select(g, h):
        gidx = idx_v[pl.ds(g * _KG, _KG)]
        for k in range(_KG):
            pltpu.make_async_copy(
                tab.at[pl.ds(0, 8), pl.ds(_MAIN, _TAIL)],
                blk_v.at[h, k], sem_t0 if h == 0 else sem_t1).wait()
        for k in range(_KG):
            idx = gidx[k]
            r = lax.rem(idx, 8)
            row = g * _KG + k
            row_v[row, pl.ds(_MAIN, 16)] = blk_v[h, k, r, pl.ds(0, 16)]
            row_v[row, pl.ds(_MAIN + 16, 16)] = blk_v[h, k, r, pl.ds(16, 16)]
            row_v[row, pl.ds(_MAIN + _TAIL - 16, 16)] = (
                blk_v[h, k, r, pl.ds(_TAIL - 16, 16)])

    def chunk_body(j, _):
        ob = pl.multiple_of(base + j * _CH, _CH)
        pltpu.sync_copy(ids.at[pl.ds(ob, _CH)], idx_v)

        @pl.when(j > 0)
        def _():
            pltpu.make_async_copy(
                row_v, out_hbm.at[pl.ds(base, _CH)], sem_w).wait()

        main_cp = pltpu.async_copy(
            tab.at[idx_v, pl.ds(0, _MAIN)], row_v.at[:, pl.ds(0, _MAIN)],
            sem_m)
        fire_tail(0, 0)
        for g in range(_NG):
            if g + 1 < _NG:
                fire_tail(g + 1, (g + 1) % 2)
            drain_select(g, g % 2)
        main_cp.wait()
        pltpu.async_copy(row_v, out_hbm.at[pl.ds(ob, _CH)], sem_w)
        return 0

    lax.fori_loop(0, _NCH, chunk_body, 0)
    pltpu.make_async_copy(
        row_v, out_hbm.at[pl.ds(base, _CH)], sem_w).wait()


_BLK = 2048
_NTOP = BATCH // _BLK
_NBLK = 2 * _NTOP


def _tc_src_body(xs_ref, we_ref, b_ref, wd_ref, o_ref, wc_ref, bc_ref):
    i = pl.program_id(0)

    @pl.when(i == 0)
    def _():
        wc_ref[...] = lax.dot_general(
            we_ref[...], wd_ref[...], (((0,), (0,)), ((), ())),
            preferred_element_type=jnp.float32)
        bc_ref[...] = lax.dot_general(
            wd_ref[...], b_ref[...], (((0,), (1,)), ((), ())),
            preferred_element_type=jnp.float32)

    o_ref[...] = lax.dot_general(
        wc_ref[...], xs_ref[:, :DIM], (((0,), (1,)), ((), ())),
        preferred_element_type=jnp.float32) + bc_ref[...]


def _tc_src(stag_src, W_enc, b2, W_dec):
    return pl.pallas_call(
        _tc_src_body,
        grid=(_NTOP,),
        in_specs=[
            pl.BlockSpec((_BLK, _SW), lambda i: (i, 0)),
            pl.BlockSpec((DIM, DIM), lambda i: (0, 0)),
            pl.BlockSpec((1, DIM), lambda i: (0, 0)),
            pl.BlockSpec((DIM, DIM), lambda i: (0, 0)),
        ],
        out_specs=pl.BlockSpec((DIM, _BLK), lambda i: (0, i)),
        out_shape=jax.ShapeDtypeStruct((DIM, 2 * BATCH), jnp.float32),
        scratch_shapes=[
            pltpu.VMEM((DIM, DIM), jnp.float32),
            pltpu.VMEM((DIM, 1), jnp.float32),
        ],
    )(stag_src, W_enc, b2, W_dec)


def _tc_tgt_body(o_in_ref, xt_ref, o_ref):
    del o_in_ref
    o_ref[...] = xt_ref[:, :DIM].T


def _tc_tgt(out_t, stag_tgt):
    return pl.pallas_call(
        _tc_tgt_body,
        grid=(_NTOP,),
        in_specs=[
            pl.BlockSpec((8, 128), lambda i: (0, 0)),
            pl.BlockSpec((_BLK, _SW), lambda i: (i, 0)),
        ],
        out_specs=pl.BlockSpec((DIM, _BLK), lambda i: (0, i + _NTOP)),
        out_shape=jax.ShapeDtypeStruct((DIM, 2 * BATCH), jnp.float32),
        input_output_aliases={0: 0},
    )(out_t, stag_tgt)


def kernel(src_id, tgt_id, emb_src, emb_tgt, W_enc, b_enc, W_dec):
    src_id = src_id.astype(jnp.int32)
    tgt_id = tgt_id.astype(jnp.int32)
    stag_src = _sc_gather(src_id, emb_src)
    stag_tgt = _sc_gather(tgt_id, emb_tgt)
    b2 = b_enc.reshape(1, DIM)
    out_t = _tc_src(stag_src, W_enc, b2, W_dec)
    out_t = _tc_tgt(out_t, stag_tgt)
    return out_t.T


# speedup vs baseline: 1.7059x; 1.0101x over previous
import functools

import jax
import jax.numpy as jnp
from jax import lax
from jax.experimental import pallas as pl
from jax.experimental.pallas import tpu as pltpu
from jax.experimental.pallas import tpu_sc as plsc

VOCAB = 100000
DIM = 300
BATCH = 16384

_NC = 2
_NS = 16
_NW = _NC * _NS
_BPW = BATCH // _NW
_CH = 128
_NCH = _BPW // _CH
_KG = 16
_NG = _CH // _KG
_MAIN = 256
_TAIL = DIM - _MAIN
_SW = 384

_mesh = plsc.VectorSubcoreMesh(core_axis_name="c", subcore_axis_name="s")


@functools.partial(
    pl.kernel,
    mesh=_mesh,
    out_type=jax.ShapeDtypeStruct((BATCH, _SW), jnp.float32),
    scratch_types=[
        pltpu.VMEM((_CH,), jnp.int32),
        pltpu.VMEM((_CH, _MAIN), jnp.float32),
        pltpu.VMEM((_CH, 128), jnp.float32),
        pltpu.VMEM((2, _KG, 8, _TAIL), jnp.float32),
        pltpu.SemaphoreType.DMA,
        pltpu.SemaphoreType.DMA,
        pltpu.SemaphoreType.DMA,
        pltpu.SemaphoreType.DMA,
    ],
)
def _sc_gather(ids, tab, out_hbm, idx_v, main_v, tail_v, blk_v,
               sem_m, sem_w, sem_t0, sem_t1):
    wid = lax.axis_index("s") * _NC + lax.axis_index("c")
    base = wid * _BPW

    def fire_tail(g, h):
        gidx = idx_v[pl.ds(g * _KG, _KG)]
        for k in range(_KG):
            idx = gidx[k]
            i8 = pl.multiple_of((idx // 8) * 8, 8)
            pltpu.async_copy(
                tab.at[pl.ds(i8, 8), pl.ds(_MAIN, _TAIL)],
                blk_v.at[h, k], sem_t0 if h == 0 else sem_t1)

    def drain_select(g, h):
        gidx = idx_v[pl.ds(g * _KG, _KG)]
        for k in range(_KG):
            pltpu.make_async_copy(
                tab.at[pl.ds(0, 8), pl.ds(_MAIN, _TAIL)],
                blk_v.at[h, k], sem_t0 if h == 0 else sem_t1).wait()
        for k in range(_KG):
            idx = gidx[k]
            r = lax.rem(idx, 8)
            row = g * _KG + k
            tail_v[row, pl.ds(0, 16)] = blk_v[h, k, r, pl.ds(0, 16)]
            tail_v[row, pl.ds(16, 16)] = blk_v[h, k, r, pl.ds(16, 16)]
            tail_v[row, pl.ds(_TAIL - 16, 16)] = (
                blk_v[h, k, r, pl.ds(_TAIL - 16, 16)])

    def chunk_body(j, _):
        ob = pl.multiple_of(base + j * _CH, _CH)
        pltpu.sync_copy(ids.at[pl.ds(ob, _CH)], idx_v)

        @pl.when(j > 0)
        def _():
            pltpu.make_async_copy(
                main_v, out_hbm.at[pl.ds(base, _CH), pl.ds(0, _MAIN)],
                sem_w).wait()
            pltpu.make_async_copy(
                tail_v, out_hbm.at[pl.ds(base, _CH), pl.ds(_MAIN, 128)],
                sem_w).wait()

        main_cp = pltpu.async_copy(
            tab.at[idx_v, pl.ds(0, _MAIN)], main_v, sem_m)
        fire_tail(0, 0)
        for g in range(_NG):
            if g + 1 < _NG:
                fire_tail(g + 1, (g + 1) % 2)
            drain_select(g, g % 2)
        main_cp.wait()
        pltpu.async_copy(main_v, out_hbm.at[pl.ds(ob, _CH), pl.ds(0, _MAIN)],
                         sem_w)
        pltpu.async_copy(tail_v,
                         out_hbm.at[pl.ds(ob, _CH), pl.ds(_MAIN, 128)],
                         sem_w)
        return 0

    lax.fori_loop(0, _NCH, chunk_body, 0)
    pltpu.make_async_copy(
        main_v, out_hbm.at[pl.ds(base, _CH), pl.ds(0, _MAIN)], sem_w).wait()
    pltpu.make_async_copy(
        tail_v, out_hbm.at[pl.ds(base, _CH), pl.ds(_MAIN, 128)], sem_w).wait()


_BLK = 2048
_NTOP = BATCH // _BLK
_NBLK = 2 * _NTOP


def _tc_src_body(xs_ref, we_ref, b_ref, wd_ref, o_ref, wc_ref, bc_ref):
    i = pl.program_id(0)

    @pl.when(i == 0)
    def _():
        wc_ref[...] = lax.dot_general(
            we_ref[...], wd_ref[...], (((0,), (0,)), ((), ())),
            preferred_element_type=jnp.float32)
        bc_ref[...] = lax.dot_general(
            wd_ref[...], b_ref[...], (((0,), (1,)), ((), ())),
            preferred_element_type=jnp.float32)

    o_ref[...] = lax.dot_general(
        wc_ref[...], xs_ref[:, :DIM], (((0,), (1,)), ((), ())),
        preferred_element_type=jnp.float32) + bc_ref[...]


def _tc_src(stag_src, W_enc, b2, W_dec):
    return pl.pallas_call(
        _tc_src_body,
        grid=(_NTOP,),
        in_specs=[
            pl.BlockSpec((_BLK, _SW), lambda i: (i, 0)),
            pl.BlockSpec((DIM, DIM), lambda i: (0, 0)),
            pl.BlockSpec((1, DIM), lambda i: (0, 0)),
            pl.BlockSpec((DIM, DIM), lambda i: (0, 0)),
        ],
        out_specs=pl.BlockSpec((DIM, _BLK), lambda i: (0, i)),
        out_shape=jax.ShapeDtypeStruct((DIM, 2 * BATCH), jnp.float32),
        scratch_shapes=[
            pltpu.VMEM((DIM, DIM), jnp.float32),
            pltpu.VMEM((DIM, 1), jnp.float32),
        ],
    )(stag_src, W_enc, b2, W_dec)


def _tc_tgt_body(o_in_ref, xt_ref, o_ref):
    del o_in_ref
    o_ref[...] = xt_ref[:, :DIM].T


def _tc_tgt(out_t, stag_tgt):
    return pl.pallas_call(
        _tc_tgt_body,
        grid=(_NTOP,),
        in_specs=[
            pl.BlockSpec((8, 128), lambda i: (0, 0)),
            pl.BlockSpec((_BLK, _SW), lambda i: (i, 0)),
        ],
        out_specs=pl.BlockSpec((DIM, _BLK), lambda i: (0, i + _NTOP)),
        out_shape=jax.ShapeDtypeStruct((DIM, 2 * BATCH), jnp.float32),
        input_output_aliases={0: 0},
    )(out_t, stag_tgt)


def kernel(src_id, tgt_id, emb_src, emb_tgt, W_enc, b_enc, W_dec):
    src_id = src_id.astype(jnp.int32)
    tgt_id = tgt_id.astype(jnp.int32)
    stag_src = _sc_gather(src_id, emb_src)
    stag_tgt = _sc_gather(tgt_id, emb_tgt)
    b2 = b_enc.reshape(1, DIM)
    out_t = _tc_src(stag_src, W_enc, b2, W_dec)
    out_t = _tc_tgt(out_t, stag_tgt)
    return out_t.T
